# Initial kernel scaffold; baseline (speedup 1.0000x reference)
import jax
import jax.numpy as jnp
from jax.experimental import pallas as pl


def kernel(embeddings, evidence_strengths, current_labels, num_clusters):
    raise NotImplementedError("write your pallas kernel here")


# speedup vs baseline: 1.9406x; 1.9406x over previous
import jax
import jax.numpy as jnp
from jax import lax
from jax.experimental import pallas as pl
from jax.experimental.pallas import tpu as pltpu

CONF_THR = 0.5
DIST_THR = 2.0
NUM_K = 512

_RAND_CENTERS = jax.random.normal(jax.random.key(42), (NUM_K, 256), jnp.float32)


def _phase_a_body(ev_ref, lab_ref, emb_ref, conf_ref, sums_ref, cnt_ref):
    i = pl.program_id(0)
    ev = ev_ref[...]
    conf = jnp.sum(ev, axis=1) * (1.0 / ev.shape[1])
    conf_ref[...] = conf
    high = conf > CONF_THR
    labs = lab_ref[...]
    b = labs.shape[0]
    kio = lax.broadcasted_iota(jnp.int32, (b, NUM_K), 1)
    w = jnp.where((labs[:, None] == kio) & high[:, None], 1.0, 0.0)

    @pl.when(i == 0)
    def _():
        sums_ref[...] = jnp.zeros_like(sums_ref)
        cnt_ref[...] = jnp.zeros_like(cnt_ref)

    sums_ref[...] += lax.dot_general(
        w, emb_ref[...], (((0,), (0,)), ((), ())),
        preferred_element_type=jnp.float32)
    cnt_ref[...] += jnp.sum(w, axis=0)[:, None]


def _phase_b_body(emb_ref, conf_ref, lab_ref, sums_ref, cnt_ref, rand_ref,
                  nl_ref, md_ref):
    counts = cnt_ref[...]
    centers = jnp.where(counts > 0.0,
                        sums_ref[...] / jnp.maximum(counts, 1.0),
                        rand_ref[...])
    c2 = jnp.sum(centers * centers, axis=1)[None, :]
    emb = emb_ref[...]
    x2 = jnp.sum(emb * emb, axis=1)[:, None]
    dot = lax.dot_general(
        emb, centers, (((1,), (1,)), ((), ())),
        preferred_element_type=jnp.float32)
    d2 = jnp.maximum(x2 + c2 - 2.0 * dot, 0.0)
    mind2 = jnp.min(d2, axis=1)
    min_d = jnp.sqrt(mind2)
    kio = lax.broadcasted_iota(jnp.int32, d2.shape, 1)
    near = jnp.min(jnp.where(d2 == mind2[:, None], kio, NUM_K), axis=1)
    near = near.astype(jnp.int32)
    reassigned = jnp.where(min_d > DIST_THR, jnp.int32(-1), near)
    hard = jnp.logical_not(conf_ref[...] > CONF_THR)
    nl_ref[...] = jnp.where(hard, reassigned, lab_ref[...])
    md_ref[...] = min_d


def kernel(embeddings, evidence_strengths, current_labels, num_clusters):
    n, d = embeddings.shape
    l = evidence_strengths.shape[1]
    ev2 = evidence_strengths.reshape(n, l)
    ba = 512
    bb = 512

    conf, sums, counts = pl.pallas_call(
        _phase_a_body,
        grid=(n // ba,),
        in_specs=[
            pl.BlockSpec((ba, l), lambda i: (i, 0)),
            pl.BlockSpec((ba,), lambda i: (i,)),
            pl.BlockSpec((ba, d), lambda i: (i, 0)),
        ],
        out_specs=[
            pl.BlockSpec((ba,), lambda i: (i,)),
            pl.BlockSpec((NUM_K, d), lambda i: (0, 0)),
            pl.BlockSpec((NUM_K, 1), lambda i: (0, 0)),
        ],
        out_shape=[
            jax.ShapeDtypeStruct((n,), jnp.float32),
            jax.ShapeDtypeStruct((NUM_K, d), jnp.float32),
            jax.ShapeDtypeStruct((NUM_K, 1), jnp.float32),
        ],
        compiler_params=pltpu.CompilerParams(
            dimension_semantics=("arbitrary",)),
    )(ev2, current_labels, embeddings)

    new_labels, min_d = pl.pallas_call(
        _phase_b_body,
        grid=(n // bb,),
        in_specs=[
            pl.BlockSpec((bb, d), lambda i: (i, 0)),
            pl.BlockSpec((bb,), lambda i: (i,)),
            pl.BlockSpec((bb,), lambda i: (i,)),
            pl.BlockSpec((NUM_K, d), lambda i: (0, 0)),
            pl.BlockSpec((NUM_K, 1), lambda i: (0, 0)),
            pl.BlockSpec((NUM_K, d), lambda i: (0, 0)),
        ],
        out_specs=[
            pl.BlockSpec((bb,), lambda i: (i,)),
            pl.BlockSpec((bb,), lambda i: (i,)),
        ],
        out_shape=[
            jax.ShapeDtypeStruct((n,), jnp.int32),
            jax.ShapeDtypeStruct((n,), jnp.float32),
        ],
        compiler_params=pltpu.CompilerParams(
            dimension_semantics=("arbitrary",)),
    )(embeddings, conf, current_labels, sums, counts, _RAND_CENTERS)

    return new_labels, min_d, conf
